# baseline (speedup 1.0000x reference)
import functools

import jax
import jax.numpy as jnp
from jax import lax
from jax.experimental import pallas as pl
from jax.experimental.pallas import tpu as pltpu
from jax.experimental.pallas import tpu_sc as plsc

N = 10000
E = 160000
IN_C = 256
HID = 512
OUT_C = 256

NC = 2
NS = 16
NW = NC * NS
PB = 64
NBT = 81
EP = NW * NBT * PB
ACC_ROWS = 10240
RPT = ACC_ROWS // NS
R = 1024
GRID = ACC_ROWS // R


DEGW = 128


@functools.cache
def _make_deg_sc(w=DEGW):
    mesh = plsc.VectorSubcoreMesh(
        core_axis_name="c", subcore_axis_name="s",
        num_cores=NC, num_subcores=NS)
    return functools.partial(
        pl.kernel,
        out_type=jax.ShapeDtypeStruct((NC, ACC_ROWS, w), jnp.float32),
        mesh=mesh,
        scratch_types=[
            pltpu.VMEM((NBT, PB), jnp.int32),
            pltpu.VMEM((PB, w), jnp.float32),
            pltpu.SemaphoreType.DMA,
            pltpu.VMEM_SHARED((ACC_ROWS, w), jnp.float32),
        ],
    )(_deg_sc_body)


def _deg_sc_body(dstp_hbm, zeros_hbm, ones_hbm, out_hbm, dst_v, ones_v, sem,
                 acc):
    ci = lax.axis_index("c")
    s = lax.axis_index("s")
    w = ci * NS + s
    pltpu.sync_copy(dstp_hbm.at[w], dst_v)
    pltpu.sync_copy(ones_hbm, ones_v)
    pltpu.sync_copy(zeros_hbm, acc.at[pl.ds(s * RPT, RPT)])
    plsc.subcore_barrier()

    def fire(j, carry):
        pltpu.async_copy(ones_v, acc.at[dst_v.at[j]], sem, add=True)
        return carry

    def drain(j, carry):
        pltpu.make_async_copy(ones_v, acc.at[dst_v.at[j]], sem).wait()
        return carry

    lax.fori_loop(0, 8, fire, 0)

    def step(j, carry):
        carry = drain(j - 8, carry)
        return fire(j, carry)

    lax.fori_loop(8, NBT, step, 0)
    lax.fori_loop(NBT - 8, NBT, drain, 0)
    plsc.subcore_barrier()
    pltpu.sync_copy(acc.at[pl.ds(s * RPT, RPT)],
                    out_hbm.at[ci, pl.ds(s * RPT, RPT)])


@functools.cache
def _make_agg_sc():
    mesh = plsc.VectorSubcoreMesh(
        core_axis_name="c", subcore_axis_name="s",
        num_cores=NC, num_subcores=NS)
    return functools.partial(
        pl.kernel,
        out_type=jax.ShapeDtypeStruct((NC, ACC_ROWS, 128), jnp.float32),
        mesh=mesh,
        scratch_types=[
            pltpu.VMEM((NBT, PB), jnp.int32),
            pltpu.VMEM((NBT, PB), jnp.int32),
            pltpu.VMEM((PB, 128), jnp.float32),
            pltpu.VMEM((PB, 128), jnp.float32),
            pltpu.VMEM((PB, 128), jnp.float32),
            pltpu.SemaphoreType.DMA,
            pltpu.SemaphoreType.DMA,
            pltpu.SemaphoreType.DMA,
            pltpu.SemaphoreType.DMA,
            pltpu.SemaphoreType.DMA,
            pltpu.SemaphoreType.DMA,
            pltpu.VMEM_SHARED((ACC_ROWS, 128), jnp.float32),
        ],
    )(_agg_sc_body)


def _agg_sc_body(g_hbm, srcp_hbm, dstp_hbm, zeros_hbm, out_hbm,
                 src_v, dst_v, b0, b1, b2,
                 gs0, gs1, gs2, ss0, ss1, ss2, acc):
    bufs = (b0, b1, b2)
    gsem = (gs0, gs1, gs2)
    ssem = (ss0, ss1, ss2)
    ci = lax.axis_index("c")
    s = lax.axis_index("s")
    w = ci * NS + s
    pltpu.sync_copy(srcp_hbm.at[w], src_v)
    pltpu.sync_copy(dstp_hbm.at[w], dst_v)

    def gather(j, b):
        pltpu.async_copy(g_hbm.at[src_v.at[j]], bufs[b], gsem[b])

    def wait_gather(j, b):
        pltpu.make_async_copy(g_hbm.at[src_v.at[j]], bufs[b], gsem[b]).wait()

    def scatter(j, b):
        pltpu.async_copy(bufs[b], acc.at[dst_v.at[j]], ssem[b], add=True)

    def wait_scatter(j, b):
        pltpu.make_async_copy(bufs[b], acc.at[dst_v.at[j]], ssem[b]).wait()

    gather(0, 0)
    gather(1, 1)
    pltpu.sync_copy(zeros_hbm, acc.at[pl.ds(s * RPT, RPT)])
    plsc.subcore_barrier()

    gather(2, 2)
    wait_gather(0, 0)
    scatter(0, 0)

    wait_scatter(0, 0)
    gather(3, 0)
    wait_gather(1, 1)
    scatter(1, 1)

    wait_scatter(1, 1)
    gather(4, 1)
    wait_gather(2, 2)
    scatter(2, 2)

    def body(jj, carry):
        for t in range(3):
            j = 3 * jj + t
            bp = (t + 2) % 3
            m = jnp.minimum(j + 2, NBT - 1)
            wait_scatter(j - 1, bp)
            gather(m, bp)
            wait_gather(j, t)
            scatter(j, t)
        return carry

    lax.fori_loop(1, NBT // 3, body, 0)
    wait_scatter(NBT - 1, 2)
    wait_gather(NBT - 1, 0)
    wait_gather(NBT - 1, 1)
    plsc.subcore_barrier()
    pltpu.sync_copy(acc.at[pl.ds(s * RPT, RPT)],
                    out_hbm.at[ci, pl.ds(s * RPT, RPT)])


def _dinv_body(degp_ref, o):
    d = degp_ref[...]
    o[...] = lax.rsqrt(d[0, :, 0:8] + d[1, :, 0:8] + 1.0)


_dinvk = pl.pallas_call(
    _dinv_body,
    grid=(GRID,),
    in_specs=[pl.BlockSpec((NC, R, DEGW), lambda i: (0, i, 0))],
    out_specs=pl.BlockSpec((R, 8), lambda i: (i, 0)),
    out_shape=jax.ShapeDtypeStruct((ACC_ROWS, 8), jnp.float32),
)


def _mm1a_body(x_ref, w1_ref, o0, o1, o2, o3):
    h = jnp.dot(x_ref[...], w1_ref[...], preferred_element_type=jnp.float32)
    o0[...] = h[:, 0:128]
    o1[...] = h[:, 128:256]
    o2[...] = h[:, 256:384]
    o3[...] = h[:, 384:512]


_mm1a = pl.pallas_call(
    _mm1a_body,
    grid=(GRID,),
    in_specs=[
        pl.BlockSpec((R, IN_C), lambda i: (i, 0)),
        pl.BlockSpec((IN_C, HID), lambda i: (0, 0)),
    ],
    out_specs=[pl.BlockSpec((R, 128), lambda i: (i, 0))] * 4,
    out_shape=[jax.ShapeDtypeStruct((N, 128), jnp.float32)] * 4,
)


def _scale_body(h0, h1, h2, h3, dinv_ref, o0, o1, o2, o3):
    dinv = dinv_ref[:, 0:1]
    for h, o in zip((h0, h1, h2, h3), (o0, o1, o2, o3)):
        o[...] = h[...] * dinv


_scale = pl.pallas_call(
    _scale_body,
    grid=(GRID,),
    in_specs=(
        [pl.BlockSpec((R, 128), lambda i: (i, 0))] * 4
        + [pl.BlockSpec((R, 8), lambda i: (i, 0))]
    ),
    out_specs=[pl.BlockSpec((R, 128), lambda i: (i, 0))] * 4,
    out_shape=[jax.ShapeDtypeStruct((N, 128), jnp.float32)] * 4,
)


def _layer2_chunk(p_ref, g_ref, dinv_ref, b1c_ref, w2c_ref, prev_ref):
    dinv = dinv_ref[:, 0:1]
    pc = p_ref[...]
    acc = pc[0] + pc[1] + g_ref[...]
    y = jnp.maximum(acc * dinv + b1c_ref[0:1, :], 0.0)
    t = jnp.dot(y, w2c_ref[...], preferred_element_type=jnp.float32)
    if prev_ref is not None:
        t = t + prev_ref[...]
    return t, dinv


def _mm2_first_body(p, g, dinv_ref, b1c, w2c, o):
    t, _ = _layer2_chunk(p, g, dinv_ref, b1c, w2c, None)
    o[...] = t


def _mm2_mid_body(p, g, prev, dinv_ref, b1c, w2c, o):
    t, _ = _layer2_chunk(p, g, dinv_ref, b1c, w2c, prev)
    o[...] = t


def _mm2_last_body(p, g, prev, dinv_ref, b1c, w2c, o0, o1):
    t, dinv = _layer2_chunk(p, g, dinv_ref, b1c, w2c, prev)
    g2o = t * dinv
    o0[...] = g2o[:, 0:128]
    o1[...] = g2o[:, 128:256]


_CHUNK_SPECS = [
    pl.BlockSpec((NC, R, 128), lambda i: (0, i, 0)),
    pl.BlockSpec((R, 128), lambda i: (i, 0)),
]
_TAIL_SPECS = [
    pl.BlockSpec((R, 8), lambda i: (i, 0)),
    pl.BlockSpec((8, 128), lambda i: (0, 0)),
    pl.BlockSpec((128, OUT_C), lambda i: (0, 0)),
]
_PREV_SPEC = pl.BlockSpec((R, OUT_C), lambda i: (i, 0))

_mm2_first = pl.pallas_call(
    _mm2_first_body,
    grid=(GRID,),
    in_specs=_CHUNK_SPECS + _TAIL_SPECS,
    out_specs=pl.BlockSpec((R, OUT_C), lambda i: (i, 0)),
    out_shape=jax.ShapeDtypeStruct((N, OUT_C), jnp.float32),
)

_mm2_mid = pl.pallas_call(
    _mm2_mid_body,
    grid=(GRID,),
    in_specs=_CHUNK_SPECS + [_PREV_SPEC] + _TAIL_SPECS,
    out_specs=pl.BlockSpec((R, OUT_C), lambda i: (i, 0)),
    out_shape=jax.ShapeDtypeStruct((N, OUT_C), jnp.float32),
)

_mm2_last = pl.pallas_call(
    _mm2_last_body,
    grid=(GRID,),
    in_specs=_CHUNK_SPECS + [_PREV_SPEC] + _TAIL_SPECS,
    out_specs=[pl.BlockSpec((R, 128), lambda i: (i, 0))] * 2,
    out_shape=[jax.ShapeDtypeStruct((N, 128), jnp.float32)] * 2,
)


def _fin_body(q, g, dinv_ref, b2c_ref, out_ref):
    dinv = dinv_ref[:, 0:1]
    qc = q[...]
    out_ref[...] = (qc[0] + qc[1] + g[...]) * dinv + b2c_ref[0:1, :]


_fin1 = pl.pallas_call(
    _fin_body,
    grid=(GRID,),
    in_specs=_CHUNK_SPECS + [
        pl.BlockSpec((R, 8), lambda i: (i, 0)),
        pl.BlockSpec((8, 128), lambda i: (0, 0)),
    ],
    out_specs=pl.BlockSpec((R, 128), lambda i: (i, 0)),
    out_shape=jax.ShapeDtypeStruct((N, 128), jnp.float32),
)


def kernel(x, edge_index, W1, b1, W2, b2):
    x = x.astype(jnp.float32)
    src = edge_index[0].astype(jnp.int32)
    dst = edge_index[1].astype(jnp.int32)
    npad = EP - E
    pad_src = (jnp.arange(npad, dtype=jnp.int32) % N)
    pad_dst = jnp.full((npad,), N, jnp.int32)
    srcp = jnp.concatenate([src, pad_src]).reshape(NW, NBT, PB)
    dstp = jnp.concatenate([dst, pad_dst]).reshape(NW, NBT, PB)

    ones_deg = jnp.ones((PB, DEGW), jnp.float32)
    zeros128 = jnp.zeros((RPT, 128), jnp.float32)

    deg_sc = _make_deg_sc()
    agg_sc = _make_agg_sc()

    degp = deg_sc(dstp, zeros128, ones_deg)
    h = _mm1a(x, W1)
    dinv = _dinvk(degp)
    g1 = _scale(*h, dinv)
    p1 = [agg_sc(gc, srcp, dstp, zeros128) for gc in g1]

    b1c = [jnp.broadcast_to(b1[128 * c:128 * (c + 1)], (8, 128))
           for c in range(4)]
    w2c = [W2[128 * c:128 * (c + 1), :] for c in range(4)]
    t = _mm2_first(p1[0], g1[0], dinv, b1c[0], w2c[0])
    t = _mm2_mid(p1[1], g1[1], t, dinv, b1c[1], w2c[1])
    t = _mm2_mid(p1[2], g1[2], t, dinv, b1c[2], w2c[2])
    g2 = _mm2_last(p1[3], g1[3], t, dinv, b1c[3], w2c[3])

    p2 = [agg_sc(gc, srcp, dstp, zeros128) for gc in g2]
    b2c = [jnp.broadcast_to(b2[128 * c:128 * (c + 1)], (8, 128))
           for c in range(2)]
    out0 = _fin1(p2[0], g2[0], dinv, b2c[0])
    out1 = _fin1(p2[1], g2[1], dinv, b2c[1])
    return jnp.concatenate([out0, out1], axis=1)


# speedup vs baseline: 1.1244x; 1.1244x over previous
import functools

import jax
import jax.numpy as jnp
from jax import lax
from jax.experimental import pallas as pl
from jax.experimental.pallas import tpu as pltpu
from jax.experimental.pallas import tpu_sc as plsc

N = 10000
E = 160000
IN_C = 256
HID = 512
OUT_C = 256

NC = 2
NS = 16
NW = NC * NS
PB = 64
NBT = 81
EP = NW * NBT * PB
ACC_ROWS = 10240
RPT = ACC_ROWS // NS
R = 1024
GRID = ACC_ROWS // R


DEGW = 128


def _zero_acc_slice(zbuf, zsem, acc, s):
    z16 = jnp.zeros((16,), jnp.float32)

    def fill(r, c):
        for k in range(8):
            zbuf[r, pl.ds(16 * k, 16)] = z16
        return c

    lax.fori_loop(0, PB, fill, 0)

    def zcopy(i, c):
        pltpu.async_copy(zbuf, acc.at[pl.ds(s * RPT + i * PB, PB)], zsem)
        return c

    lax.fori_loop(0, RPT // PB, zcopy, 0)

    def zdrain(i, c):
        pltpu.make_async_copy(
            zbuf, acc.at[pl.ds(s * RPT + i * PB, PB)], zsem).wait()
        return c

    lax.fori_loop(0, RPT // PB, zdrain, 0)


@functools.cache
def _make_deg_sc(w=DEGW):
    mesh = plsc.VectorSubcoreMesh(
        core_axis_name="c", subcore_axis_name="s",
        num_cores=NC, num_subcores=NS)
    return functools.partial(
        pl.kernel,
        out_type=jax.ShapeDtypeStruct((NC, ACC_ROWS, w), jnp.float32),
        mesh=mesh,
        scratch_types=[
            pltpu.VMEM((NBT, PB), jnp.int32),
            pltpu.VMEM((PB, w), jnp.float32),
            pltpu.SemaphoreType.DMA,
            pltpu.VMEM_SHARED((ACC_ROWS, w), jnp.float32),
        ],
    )(_deg_sc_body)


def _deg_sc_body(dstp_hbm, ones_hbm, out_hbm, dst_v, ones_v, sem, acc):
    ci = lax.axis_index("c")
    s = lax.axis_index("s")
    w = ci * NS + s
    pltpu.sync_copy(dstp_hbm.at[w], dst_v)
    _zero_acc_slice(ones_v, sem, acc, s)
    pltpu.sync_copy(ones_hbm, ones_v)
    plsc.subcore_barrier()

    def fire(j, carry):
        pltpu.async_copy(ones_v, acc.at[dst_v.at[j]], sem, add=True)
        return carry

    def drain(j, carry):
        pltpu.make_async_copy(ones_v, acc.at[dst_v.at[j]], sem).wait()
        return carry

    lax.fori_loop(0, 8, fire, 0)

    def step(j, carry):
        carry = drain(j - 8, carry)
        return fire(j, carry)

    lax.fori_loop(8, NBT, step, 0)
    lax.fori_loop(NBT - 8, NBT, drain, 0)
    plsc.subcore_barrier()
    pltpu.sync_copy(acc.at[pl.ds(s * RPT, RPT)],
                    out_hbm.at[ci, pl.ds(s * RPT, RPT)])


@functools.cache
def _make_agg_sc():
    mesh = plsc.VectorSubcoreMesh(
        core_axis_name="c", subcore_axis_name="s",
        num_cores=NC, num_subcores=NS)
    return functools.partial(
        pl.kernel,
        out_type=jax.ShapeDtypeStruct((NC, ACC_ROWS, 128), jnp.float32),
        mesh=mesh,
        scratch_types=[
            pltpu.VMEM((NBT, PB), jnp.int32),
            pltpu.VMEM((NBT, PB), jnp.int32),
            pltpu.VMEM((PB, 128), jnp.float32),
            pltpu.VMEM((PB, 128), jnp.float32),
            pltpu.VMEM((PB, 128), jnp.float32),
            pltpu.SemaphoreType.DMA,
            pltpu.SemaphoreType.DMA,
            pltpu.SemaphoreType.DMA,
            pltpu.SemaphoreType.DMA,
            pltpu.SemaphoreType.DMA,
            pltpu.SemaphoreType.DMA,
            pltpu.VMEM_SHARED((ACC_ROWS, 128), jnp.float32),
        ],
    )(_agg_sc_body)


def _agg_sc_body(g_hbm, srcp_hbm, dstp_hbm, out_hbm,
                 src_v, dst_v, b0, b1, b2,
                 gs0, gs1, gs2, ss0, ss1, ss2, acc):
    bufs = (b0, b1, b2)
    gsem = (gs0, gs1, gs2)
    ssem = (ss0, ss1, ss2)
    ci = lax.axis_index("c")
    s = lax.axis_index("s")
    w = ci * NS + s
    pltpu.sync_copy(srcp_hbm.at[w], src_v)
    pltpu.sync_copy(dstp_hbm.at[w], dst_v)

    def gather(j, b):
        pltpu.async_copy(g_hbm.at[src_v.at[j]], bufs[b], gsem[b])

    def wait_gather(j, b):
        pltpu.make_async_copy(g_hbm.at[src_v.at[j]], bufs[b], gsem[b]).wait()

    def scatter(j, b):
        pltpu.async_copy(bufs[b], acc.at[dst_v.at[j]], ssem[b], add=True)

    def wait_scatter(j, b):
        pltpu.make_async_copy(bufs[b], acc.at[dst_v.at[j]], ssem[b]).wait()

    gather(0, 0)
    gather(1, 1)
    _zero_acc_slice(b2, ss2, acc, s)
    plsc.subcore_barrier()

    gather(2, 2)
    wait_gather(0, 0)
    scatter(0, 0)

    wait_scatter(0, 0)
    gather(3, 0)
    wait_gather(1, 1)
    scatter(1, 1)

    wait_scatter(1, 1)
    gather(4, 1)
    wait_gather(2, 2)
    scatter(2, 2)

    def body(jj, carry):
        for t in range(3):
            j = 3 * jj + t
            bp = (t + 2) % 3
            m = jnp.minimum(j + 2, NBT - 1)
            wait_scatter(j - 1, bp)
            gather(m, bp)
            wait_gather(j, t)
            scatter(j, t)
        return carry

    lax.fori_loop(1, NBT // 3, body, 0)
    wait_scatter(NBT - 1, 2)
    wait_gather(NBT - 1, 0)
    wait_gather(NBT - 1, 1)
    plsc.subcore_barrier()
    pltpu.sync_copy(acc.at[pl.ds(s * RPT, RPT)],
                    out_hbm.at[ci, pl.ds(s * RPT, RPT)])


def _dinv_block(degp):
    d = degp[...]
    return lax.rsqrt(d[0, :, 0:1] + d[1, :, 0:1] + 1.0)


def _mm1_body(x_ref, w1_ref, degp_ref, o0, o1, o2, o3):
    dinv = _dinv_block(degp_ref)
    h = jnp.dot(x_ref[...], w1_ref[...], preferred_element_type=jnp.float32)
    g = h * dinv
    o0[...] = g[:, 0:128]
    o1[...] = g[:, 128:256]
    o2[...] = g[:, 256:384]
    o3[...] = g[:, 384:512]


_mm1 = pl.pallas_call(
    _mm1_body,
    grid=(GRID,),
    in_specs=[
        pl.BlockSpec((R, IN_C), lambda i: (i, 0)),
        pl.BlockSpec((IN_C, HID), lambda i: (0, 0)),
        pl.BlockSpec((NC, R, DEGW), lambda i: (0, i, 0)),
    ],
    out_specs=[pl.BlockSpec((R, 128), lambda i: (i, 0))] * 4,
    out_shape=[jax.ShapeDtypeStruct((N, 128), jnp.float32)] * 4,
)


def _mm2_body(p0, p1, p2, p3, g0, g1, g2, g3, degp_ref, b1_ref, w2_ref,
              o0, o1):
    dinv = _dinv_block(degp_ref)
    h2 = jnp.zeros((R, OUT_C), jnp.float32)
    for c, (p, g) in enumerate(zip((p0, p1, p2, p3), (g0, g1, g2, g3))):
        pc = p[...]
        acc = pc[0] + pc[1] + g[...]
        y = jnp.maximum(acc * dinv + b1_ref[0:1, 128 * c:128 * (c + 1)], 0.0)
        h2 = h2 + jnp.dot(y, w2_ref[128 * c:128 * (c + 1), :],
                          preferred_element_type=jnp.float32)
    g2o = h2 * dinv
    o0[...] = g2o[:, 0:128]
    o1[...] = g2o[:, 128:256]


_mm2 = pl.pallas_call(
    _mm2_body,
    grid=(GRID,),
    in_specs=(
        [pl.BlockSpec((NC, R, 128), lambda i: (0, i, 0))] * 4
        + [pl.BlockSpec((R, 128), lambda i: (i, 0))] * 4
        + [
            pl.BlockSpec((NC, R, DEGW), lambda i: (0, i, 0)),
            pl.BlockSpec((8, HID), lambda i: (0, 0)),
            pl.BlockSpec((HID, OUT_C), lambda i: (0, 0)),
        ]
    ),
    out_specs=[pl.BlockSpec((R, 128), lambda i: (i, 0))] * 2,
    out_shape=[jax.ShapeDtypeStruct((N, 128), jnp.float32)] * 2,
)


def _fin_body(q0, q1, g0, g1, degp_ref, b2_ref, out_ref):
    dinv = _dinv_block(degp_ref)
    cols = []
    for c, (q, g) in enumerate(zip((q0, q1), (g0, g1))):
        qc = q[...]
        acc = qc[0] + qc[1] + g[...]
        cols.append(acc * dinv + b2_ref[0:1, 128 * c:128 * (c + 1)])
    out_ref[...] = jnp.concatenate(cols, axis=1)


_fin = pl.pallas_call(
    _fin_body,
    grid=(GRID,),
    in_specs=(
        [pl.BlockSpec((NC, R, 128), lambda i: (0, i, 0))] * 2
        + [pl.BlockSpec((R, 128), lambda i: (i, 0))] * 2
        + [
            pl.BlockSpec((NC, R, DEGW), lambda i: (0, i, 0)),
            pl.BlockSpec((8, OUT_C), lambda i: (0, 0)),
        ]
    ),
    out_specs=pl.BlockSpec((R, OUT_C), lambda i: (i, 0)),
    out_shape=jax.ShapeDtypeStruct((N, OUT_C), jnp.float32),
)


def kernel(x, edge_index, W1, b1, W2, b2):
    x = x.astype(jnp.float32)
    src = edge_index[0].astype(jnp.int32)
    dst = edge_index[1].astype(jnp.int32)
    npad = EP - E
    pad_src = (jnp.arange(npad, dtype=jnp.int32) % N)
    pad_dst = jnp.full((npad,), N, jnp.int32)
    srcp = jnp.concatenate([src, pad_src]).reshape(NW, NBT, PB)
    dstp = jnp.concatenate([dst, pad_dst]).reshape(NW, NBT, PB)

    ones_deg = jnp.ones((PB, DEGW), jnp.float32)

    deg_sc = _make_deg_sc()
    agg_sc = _make_agg_sc()
    degp = deg_sc(dstp, ones_deg)
    g1 = _mm1(x, W1, degp)
    p1 = [agg_sc(gc, srcp, dstp) for gc in g1]
    g2 = _mm2(*p1, *g1, degp, jnp.broadcast_to(b1, (8, HID)), W2)
    p2 = [agg_sc(gc, srcp, dstp) for gc in g2]
    out = _fin(*p2, *g2, degp, jnp.broadcast_to(b2, (8, OUT_C)))
    return out
